# baseline (speedup 1.0000x reference)
import functools

import jax
import jax.numpy as jnp
from jax.experimental import pallas as pl
from jax.experimental.pallas import tpu as pltpu

B, N_MELS, T = 16, 128, 2048
D_MODEL, CODE_DIM, K_CODES = 512, 64, 1024
TQ = T // 4
BOS_TOKEN = K_CODES
EOS_TOKEN = K_CODES + 1
PAD_TOKEN = K_CODES + 2


def _encoder_body(x_ref, w1_ref, b1_ref, w2_ref, b2_ref, wp_ref, bp_ref,
                  cbt_ref, len_ref, tok_ref, lens_ref, xt_ref, h1_ref):
    f32 = jnp.float32
    xt_ref[...] = x_ref[0].T
    E = xt_ref[0::2, :]
    O = xt_ref[1::2, :]
    zr = jnp.zeros((1, N_MELS), f32)
    Om = jnp.concatenate([zr, O[:-1, :]], axis=0)
    Ep = jnp.concatenate([E[1:, :], zr], axis=0)

    def mm(a, b):
        return jnp.dot(a, b, preferred_element_type=f32)

    h1 = (mm(Om, w1_ref[0]) + mm(E, w1_ref[1]) + mm(O, w1_ref[2])
          + mm(Ep, w1_ref[3]) + b1_ref[...])
    h1 = jax.nn.gelu(h1)
    for c in range(D_MODEL // N_MELS):
        h1_ref[c] = h1[:, c * N_MELS:(c + 1) * N_MELS]
    E1 = jnp.concatenate([h1_ref[c, 0::2, :] for c in range(4)], axis=1)
    O1 = jnp.concatenate([h1_ref[c, 1::2, :] for c in range(4)], axis=1)
    zd = jnp.zeros((1, D_MODEL), f32)
    O1m = jnp.concatenate([zd, O1[:-1, :]], axis=0)
    E1p = jnp.concatenate([E1[1:, :], zd], axis=0)
    h2 = (mm(O1m, w2_ref[0]) + mm(E1, w2_ref[1]) + mm(O1, w2_ref[2])
          + mm(E1p, w2_ref[3]) + b2_ref[...])
    h2 = jax.nn.gelu(h2)

    z = mm(h2, wp_ref[...]) + bp_ref[...]
    cbt = cbt_ref[...]
    cb2 = jnp.sum(cbt * cbt, axis=0, keepdims=True)
    z_aug = jnp.concatenate([z, jnp.ones((TQ, 1), f32)], axis=1)
    cbt_aug = jnp.concatenate([cbt, -0.5 * cb2], axis=0)
    s = mm(z_aug, cbt_aug)
    smax = jnp.max(s, axis=1, keepdims=True)
    kio = jax.lax.broadcasted_iota(jnp.int32, (TQ, K_CODES), 1)
    idx = jnp.min(jnp.where(s == smax, kio, K_CODES), axis=1)

    seq_len = len_ref[pl.program_id(0), 0] // 4
    jcol = jax.lax.broadcasted_iota(jnp.int32, (TQ + 2, 1), 0)
    toks_ext = jnp.concatenate(
        [jnp.full((1, 1), BOS_TOKEN, jnp.int32),
         idx.astype(jnp.int32)[:, None],
         jnp.full((1, 1), PAD_TOKEN, jnp.int32)], axis=0)
    p = jcol - 1
    fixed = jnp.where(p == seq_len, EOS_TOKEN,
                      jnp.where(p > seq_len, PAD_TOKEN, toks_ext))
    tok_ref[0] = jnp.where(jcol == 0, BOS_TOKEN, fixed)
    lens_ref[0] = jnp.full((1, 1), seq_len + 2, jnp.int32)


@jax.jit
def kernel(mel_spec, mel_spec_lengths, w1, b1, w2, b2, wp, bp, codebook):
    w1t = jnp.transpose(w1, (2, 1, 0))
    w2t = jnp.transpose(w2, (2, 1, 0))
    b1r = b1[None, :]
    b2r = b2[None, :]
    bpr = bp[None, :]
    cbt = codebook.T
    lens_i32 = mel_spec_lengths.astype(jnp.int32)[:, None]

    tokens_col, lengths = pl.pallas_call(
        _encoder_body,
        grid=(B,),
        in_specs=[
            pl.BlockSpec((1, N_MELS, T), lambda b: (b, 0, 0)),
            pl.BlockSpec((4, N_MELS, D_MODEL), lambda b: (0, 0, 0)),
            pl.BlockSpec((1, D_MODEL), lambda b: (0, 0)),
            pl.BlockSpec((4, D_MODEL, D_MODEL), lambda b: (0, 0, 0)),
            pl.BlockSpec((1, D_MODEL), lambda b: (0, 0)),
            pl.BlockSpec((D_MODEL, CODE_DIM), lambda b: (0, 0)),
            pl.BlockSpec((1, CODE_DIM), lambda b: (0, 0)),
            pl.BlockSpec((CODE_DIM, K_CODES), lambda b: (0, 0)),
            pl.BlockSpec((B, 1), lambda b: (0, 0)),
        ],
        out_specs=[
            pl.BlockSpec((1, TQ + 2, 1), lambda b: (b, 0, 0)),
            pl.BlockSpec((1, 1, 1), lambda b: (b, 0, 0)),
        ],
        out_shape=[
            jax.ShapeDtypeStruct((B, TQ + 2, 1), jnp.int32),
            jax.ShapeDtypeStruct((B, 1, 1), jnp.int32),
        ],
        scratch_shapes=[
            pltpu.VMEM((T, N_MELS), jnp.float32),
            pltpu.VMEM((4, T // 2, N_MELS), jnp.float32),
        ],
    )(mel_spec, w1t, b1r, w2t, b2r, wp, bpr, cbt, lens_i32)

    return tokens_col.reshape(B, TQ + 2), lengths.reshape(B)
